# baseline (speedup 1.0000x reference)
import functools

import jax
import jax.numpy as jnp
from jax import lax
from jax.experimental import pallas as pl
from jax.experimental.pallas import tpu as pltpu
from jax.experimental.pallas import tpu_sc as plsc

N = 10000
E = 320000
F = 128
H = 8
HD = 16
ALPHA = 0.2

NC = 2
NS = 16
NW = NC * NS
EPT = E // NW
CHUNK = 80
NCH = EPT // CHUNK
ACHUNK = 40
ANCH = EPT // ACHUNK
NB = 104
RPS = 624
RPS_LAST = N - 15 * RPS


def _node_slab_copy(sid, src_fn, dst_fn):
    off = pl.multiple_of(sid * RPS, 8)

    @pl.when(sid < NS - 1)
    def _():
        pltpu.sync_copy(src_fn(off, RPS), dst_fn(off, RPS))

    @pl.when(sid == NS - 1)
    def _():
        pltpu.sync_copy(src_fn(15 * RPS, RPS_LAST), dst_fn(15 * RPS, RPS_LAST))

BN = 400

_GDN = lax.GatherDimensionNumbers(
    offset_dims=(), collapsed_slice_dims=(0,), start_index_map=(0,))


def _lane_bcast(row, j):
    idx = jnp.full((HD, 1), j, jnp.int32)
    return lax.gather(row, idx, _GDN, (1,),
                      mode=lax.GatherScatterMode.PROMISE_IN_BOUNDS)


def _tc_head_body(x_ref, wall_ref, a1_ref, a2_ref, h_ref, s1_ref, s2_ref):
    h = jnp.dot(x_ref[...], wall_ref[...], preferred_element_type=jnp.float32)
    h_ref[...] = h
    s1_ref[...] = jnp.dot(h, a1_ref[...], preferred_element_type=jnp.float32)
    s2_ref[...] = jnp.dot(h, a2_ref[...], preferred_element_type=jnp.float32)


def _tc_head(x, wall, a1t, a2t, interpret=False):
    return pl.pallas_call(
        _tc_head_body,
        grid=(N // BN,),
        in_specs=[
            pl.BlockSpec((BN, F), lambda i: (i, 0)),
            pl.BlockSpec((F, F), lambda i: (0, 0)),
            pl.BlockSpec((F, HD), lambda i: (0, 0)),
            pl.BlockSpec((F, HD), lambda i: (0, 0)),
        ],
        out_specs=[
            pl.BlockSpec((BN, F), lambda i: (i, 0)),
            pl.BlockSpec((BN, HD), lambda i: (i, 0)),
            pl.BlockSpec((BN, HD), lambda i: (i, 0)),
        ],
        out_shape=[
            jax.ShapeDtypeStruct((N, F), jnp.float32),
            jax.ShapeDtypeStruct((N, HD), jnp.float32),
            jax.ShapeDtypeStruct((N, HD), jnp.float32),
        ],
        interpret=interpret,
    )(x, wall, a1t, a2t)


def _sc_scores(src_r, tgt_r, s1t, s2t, mt, z16):
    mesh = plsc.VectorSubcoreMesh(core_axis_name="c", subcore_axis_name="s")

    @functools.partial(
        pl.kernel,
        out_type=(
            jax.ShapeDtypeStruct((E, HD), jnp.float32),
            jax.ShapeDtypeStruct((N, HD), jnp.float32),
            jax.ShapeDtypeStruct((N, HD), jnp.float32),
        ),
        mesh=mesh,
        scratch_types=[
            pltpu.VMEM((NCH, CHUNK), jnp.int32),
            pltpu.VMEM((NCH, CHUNK), jnp.int32),
            pltpu.VMEM((3, CHUNK, HD), jnp.float32),
            pltpu.VMEM((3, CHUNK, HD), jnp.float32),
            pltpu.VMEM((3, CHUNK, HD), jnp.float32),
            pltpu.VMEM((HD,), jnp.float32),
            pltpu.VMEM_SHARED((N, HD), jnp.float32),
            (pltpu.SemaphoreType.DMA,) * 3,
            (pltpu.SemaphoreType.DMA,) * 3,
            (pltpu.SemaphoreType.DMA,) * 3,
        ],
        compiler_params=pltpu.CompilerParams(use_tc_tiling_on_sc=False),
    )
    def k(src_hbm, tgt_hbm, s1_hbm, s2_hbm, mt_hbm, z_hbm,
          ex_hbm, n0_hbm, n1_hbm,
          src_all, tgt_all, gs2, gt2, ex2, mt_v, norm_sh, semg, semw,
          semsc):
        cid = lax.axis_index("c")
        sid = lax.axis_index("s")
        wid = sid * NC + cid
        _node_slab_copy(sid, lambda o, n: z_hbm.at[pl.ds(0, n)],
                        lambda o, n: norm_sh.at[pl.ds(o, n)])
        pltpu.sync_copy(mt_hbm, mt_v)
        pltpu.sync_copy(src_hbm.at[wid], src_all)
        pltpu.sync_copy(tgt_hbm.at[wid], tgt_all)
        plsc.subcore_barrier()
        mt_row = mt_v[...]
        base = wid * EPT
        gs = [gs2.at[0], gs2.at[1], gs2.at[2]]
        gt = [gt2.at[0], gt2.at[1], gt2.at[2]]
        exv = [ex2.at[0], ex2.at[1], ex2.at[2]]

        def descs(slot, ci):
            return (
                pltpu.make_async_copy(s1_hbm.at[src_all.at[ci]], gs[slot],
                                      semg[slot]),
                pltpu.make_async_copy(s2_hbm.at[tgt_all.at[ci]], gt[slot],
                                      semg[slot]),
            )

        def wr_descs(slot, ci):
            off = pl.multiple_of(base + ci * CHUNK, 8)
            return (
                pltpu.make_async_copy(exv[slot], ex_hbm.at[pl.ds(off, CHUNK)],
                                      semw[slot]),
                pltpu.make_async_copy(exv[slot], norm_sh.at[tgt_all.at[ci]],
                                      semsc[slot]),
            )

        def start(slot, ci):
            d1, d2 = descs(slot, ci)
            d1.start()
            d2.start()

        def work(slot, ci):
            d1, d2 = descs(slot, ci)
            d1.wait()
            d2.wait()
            dw, dsc = wr_descs(slot, ci)

            @pl.when(jnp.asarray(ci, jnp.int32) >= 3)
            def _():
                dw.wait()
                dsc.wait()

            g_s = gs[slot]
            g_t = gt[slot]
            e_v = exv[slot]
            for i in range(CHUNK):
                s = g_s[i, :] + g_t[i, :]
                s = jnp.where(s > 0, s, ALPHA * s)
                e_v[i, :] = jnp.exp(s - mt_row)
            dw.start()
            dsc.start(add=True)

        start(0, 0)
        start(1, 1)
        ntrip = (NCH - 2) // 3

        def triple(i, carry):
            c = 3 * i
            for kk in range(3):
                work(kk, c + kk)
                start((kk + 2) % 3, c + kk + 2)
            return carry

        lax.fori_loop(0, ntrip, triple, 0)
        for c in range(3 * ntrip, NCH):
            work(c % 3, c)
            if c + 2 < NCH:
                start((c + 2) % 3, c + 2)
        for slot in range(3):
            dw, dsc = wr_descs(slot, 0)
            dw.wait()
            dsc.wait()
        plsc.subcore_barrier()

        @pl.when(cid == 0)
        def _():
            _node_slab_copy(sid, lambda o, n: norm_sh.at[pl.ds(o, n)],
                            lambda o, n: n0_hbm.at[pl.ds(o, n)])

        @pl.when(cid == 1)
        def _():
            _node_slab_copy(sid, lambda o, n: norm_sh.at[pl.ds(o, n)],
                            lambda o, n: n1_hbm.at[pl.ds(o, n)])

    return k(src_r, tgt_r, s1t, s2t, mt, z16)


def _sc_aggregate(src_r, tgt_r, ex, n0, n1, h, z128):
    mesh = plsc.VectorSubcoreMesh(core_axis_name="c", subcore_axis_name="s")

    @functools.partial(
        pl.kernel,
        out_type=(
            jax.ShapeDtypeStruct((N, F), jnp.float32),
            jax.ShapeDtypeStruct((N, F), jnp.float32),
            jax.ShapeDtypeStruct((N, HD), jnp.float32),
        ),
        mesh=mesh,
        scratch_types=[
            pltpu.VMEM((ANCH, ACHUNK), jnp.int32),
            pltpu.VMEM((ANCH, ACHUNK), jnp.int32),
            pltpu.VMEM((3, ACHUNK, F), jnp.float32),
            pltpu.VMEM((3, ACHUNK, HD), jnp.float32),
            pltpu.VMEM((3, ACHUNK, HD), jnp.float32),
            pltpu.VMEM((NB, HD), jnp.float32),
            pltpu.VMEM((NB, HD), jnp.float32),
            pltpu.VMEM_SHARED((N, F), jnp.float32),
            (pltpu.SemaphoreType.DMA,) * 3,
            (pltpu.SemaphoreType.DMA,) * 3,
            (pltpu.SemaphoreType.DMA,) * 3,
            (pltpu.SemaphoreType.DMA,) * 3,
        ],
        compiler_params=pltpu.CompilerParams(use_tc_tiling_on_sc=False),
    )
    def k(src_hbm, tgt_hbm, ex_hbm, n0_hbm, n1_hbm, h_hbm, z_hbm,
          p0_hbm, p1_hbm, rn_hbm,
          src_all, tgt_all, hb3, g03, ex3, nb0, nb1, out_sh,
          semh, semn, seme, semsc):
        cid = lax.axis_index("c")
        sid = lax.axis_index("s")
        wid = sid * NC + cid
        _node_slab_copy(sid, lambda o, n: z_hbm.at[pl.ds(0, n)],
                        lambda o, n: out_sh.at[pl.ds(o, n)])
        pltpu.sync_copy(src_hbm.at[wid], src_all)
        pltpu.sync_copy(tgt_hbm.at[wid], tgt_all)

        def rn_block(off, nrows):
            pltpu.sync_copy(n0_hbm.at[pl.ds(off, nrows)],
                            nb0.at[pl.ds(0, nrows)])
            pltpu.sync_copy(n1_hbm.at[pl.ds(off, nrows)],
                            nb1.at[pl.ds(0, nrows)])
            for r in range(nrows):
                nb0[r, :] = 1.0 / (nb0[r, :] + nb1[r, :])
            pltpu.sync_copy(nb0.at[pl.ds(0, nrows)],
                            rn_hbm.at[pl.ds(off, nrows)])

        def rn_loop(b, carry):
            rn_block(pl.multiple_of(sid * RPS + b * NB, 8), NB)
            return carry

        lax.fori_loop(0, RPS // NB, rn_loop, 0)

        @pl.when(sid == NS - 1)
        def _():
            rn_block(15 * RPS + RPS // NB * NB, RPS_LAST - RPS)

        plsc.subcore_barrier()
        base = wid * EPT
        hb = [hb3.at[0], hb3.at[1], hb3.at[2]]
        g0 = [g03.at[0], g03.at[1], g03.at[2]]
        exv = [ex3.at[0], ex3.at[1], ex3.at[2]]

        def descs(slot, ci):
            off = pl.multiple_of(base + ci * ACHUNK, 8)
            return (
                pltpu.make_async_copy(h_hbm.at[src_all.at[ci]], hb[slot],
                                      semh[slot]),
                pltpu.make_async_copy(rn_hbm.at[tgt_all.at[ci]], g0[slot],
                                      semn[slot]),
                pltpu.make_async_copy(ex_hbm.at[pl.ds(off, ACHUNK)],
                                      exv[slot], seme[slot]),
            )

        def sc_desc(slot, ci):
            return pltpu.make_async_copy(hb[slot], out_sh.at[tgt_all.at[ci]],
                                         semsc[slot])

        def start(slot, ci):
            @pl.when(jnp.asarray(ci, jnp.int32) >= 3)
            def _():
                sc_desc(slot, ci).wait()
            for d in descs(slot, ci):
                d.start()

        def work(slot, ci):
            dh, d0, de = descs(slot, ci)
            d0.wait()
            de.wait()
            dh.wait()
            e_v = exv[slot]
            g_0 = g0[slot]
            h_b = hb[slot]
            for i in range(ACHUNK):
                row = e_v[i, :] * g_0[i, :]
                for j in range(H):
                    a_b = _lane_bcast(row, j)
                    h_b[i, pl.ds(j * HD, HD)] = (
                        h_b[i, pl.ds(j * HD, HD)] * a_b)
            sc_desc(slot, ci).start(add=True)

        start(0, 0)
        start(1, 1)
        ntrip = (ANCH - 2) // 3

        def triple(i, carry):
            c = 3 * i
            for kk in range(3):
                work(kk, c + kk)
                start((kk + 2) % 3, c + kk + 2)
            return carry

        lax.fori_loop(0, ntrip, triple, 0)
        for c in range(3 * ntrip, ANCH):
            work(c % 3, c)
            if c + 2 < ANCH:
                start((c + 2) % 3, c + 2)
        for slot in range(3):
            pltpu.make_async_copy(hb[slot], out_sh.at[tgt_all.at[0]],
                                  semsc[slot]).wait()
        plsc.subcore_barrier()

        @pl.when(cid == 0)
        def _():
            _node_slab_copy(sid, lambda o, n: out_sh.at[pl.ds(o, n)],
                            lambda o, n: p0_hbm.at[pl.ds(o, n)])

        @pl.when(cid == 1)
        def _():
            _node_slab_copy(sid, lambda o, n: out_sh.at[pl.ds(o, n)],
                            lambda o, n: p1_hbm.at[pl.ds(o, n)])

    return k(src_r, tgt_r, ex, n0, n1, h, z128)


def _tc_combine_body(p0_ref, p1_ref, b_ref, o_ref):
    o_ref[...] = p0_ref[...] + p1_ref[...] + b_ref[...]


def _tc_combine(p0, p1, bias, interpret=False):
    return pl.pallas_call(
        _tc_combine_body,
        grid=(N // BN,),
        in_specs=[
            pl.BlockSpec((BN, F), lambda i: (i, 0)),
            pl.BlockSpec((BN, F), lambda i: (i, 0)),
            pl.BlockSpec((1, F), lambda i: (0, 0)),
        ],
        out_specs=pl.BlockSpec((BN, F), lambda i: (i, 0)),
        out_shape=jax.ShapeDtypeStruct((N, F), jnp.float32),
        interpret=interpret,
    )(p0, p1, bias.reshape(1, F))


def kernel(x, edge_index, W, a, bias):
    src = edge_index[0].reshape(NW, NCH, CHUNK)
    tgt = edge_index[1].reshape(NW, NCH, CHUNK)
    src_a = edge_index[0].reshape(NW, ANCH, ACHUNK)
    tgt_a = edge_index[1].reshape(NW, ANCH, ACHUNK)
    wall = jnp.transpose(W, (1, 0, 2)).reshape(F, F)
    eye8 = jnp.eye(H, dtype=jnp.float32)
    a1 = a[:, :HD, 0]
    a2 = a[:, HD:, 0]
    A1 = jnp.einsum('gk,hg->hkg', a1, eye8).reshape(F, H)
    A2 = jnp.einsum('gk,hg->hkg', a2, eye8).reshape(F, H)
    a1t = jnp.concatenate([A1, A1], axis=1)
    a2t = jnp.concatenate([A2, A2], axis=1)

    h, s1t, s2t = _tc_head(x, wall, a1t, a2t)
    mt = jnp.maximum(jnp.max(s1t, axis=0) + jnp.max(s2t, axis=0), 0.0)

    z16 = jnp.zeros((RPS_LAST, HD), jnp.float32)
    ex, n0, n1 = _sc_scores(src, tgt, s1t, s2t, mt, z16)

    z128 = jnp.zeros((RPS_LAST, F), jnp.float32)
    p0, p1, _ = _sc_aggregate(src_a, tgt_a, ex, n0, n1, h, z128)

    return _tc_combine(p0, p1, bias)


# speedup vs baseline: 1.0016x; 1.0016x over previous
import functools

import jax
import jax.numpy as jnp
from jax import lax
from jax.experimental import pallas as pl
from jax.experimental.pallas import tpu as pltpu
from jax.experimental.pallas import tpu_sc as plsc

N = 10000
E = 320000
F = 128
H = 8
HD = 16
ALPHA = 0.2

NC = 2
NS = 16
NW = NC * NS
EPT = E // NW
CHUNK = 80
NCH = EPT // CHUNK
ACHUNK = 40
ANCH = EPT // ACHUNK
NB = 104
RPS = 624
RPS_LAST = N - 15 * RPS


def _node_slab_copy(sid, src_fn, dst_fn):
    off = pl.multiple_of(sid * RPS, 8)

    @pl.when(sid < NS - 1)
    def _():
        pltpu.sync_copy(src_fn(off, RPS), dst_fn(off, RPS))

    @pl.when(sid == NS - 1)
    def _():
        pltpu.sync_copy(src_fn(15 * RPS, RPS_LAST), dst_fn(15 * RPS, RPS_LAST))

BN = 400

_GDN = lax.GatherDimensionNumbers(
    offset_dims=(), collapsed_slice_dims=(0,), start_index_map=(0,))


def _lane_bcast(row, j):
    idx = jnp.full((HD, 1), j, jnp.int32)
    return lax.gather(row, idx, _GDN, (1,),
                      mode=lax.GatherScatterMode.PROMISE_IN_BOUNDS)


def _tc_head_body(x_ref, wall_ref, a1_ref, a2_ref, h_ref, s1_ref, s2_ref):
    h = jnp.dot(x_ref[...], wall_ref[...], preferred_element_type=jnp.float32)
    h_ref[...] = h
    s1_ref[...] = jnp.dot(h, a1_ref[...], preferred_element_type=jnp.float32)
    s2_ref[...] = jnp.dot(h, a2_ref[...], preferred_element_type=jnp.float32)


def _tc_head(x, wall, a1t, a2t, interpret=False):
    return pl.pallas_call(
        _tc_head_body,
        grid=(N // BN,),
        in_specs=[
            pl.BlockSpec((BN, F), lambda i: (i, 0)),
            pl.BlockSpec((F, F), lambda i: (0, 0)),
            pl.BlockSpec((F, HD), lambda i: (0, 0)),
            pl.BlockSpec((F, HD), lambda i: (0, 0)),
        ],
        out_specs=[
            pl.BlockSpec((BN, F), lambda i: (i, 0)),
            pl.BlockSpec((BN, HD), lambda i: (i, 0)),
            pl.BlockSpec((BN, HD), lambda i: (i, 0)),
        ],
        out_shape=[
            jax.ShapeDtypeStruct((N, F), jnp.float32),
            jax.ShapeDtypeStruct((N, HD), jnp.float32),
            jax.ShapeDtypeStruct((N, HD), jnp.float32),
        ],
        interpret=interpret,
    )(x, wall, a1t, a2t)


def _sc_scores(src_r, tgt_r, s1t, s2t, mt, z16):
    mesh = plsc.VectorSubcoreMesh(core_axis_name="c", subcore_axis_name="s")

    @functools.partial(
        pl.kernel,
        out_type=(
            jax.ShapeDtypeStruct((E, HD), jnp.float32),
            jax.ShapeDtypeStruct((N, HD), jnp.float32),
            jax.ShapeDtypeStruct((N, HD), jnp.float32),
        ),
        mesh=mesh,
        scratch_types=[
            pltpu.VMEM((NCH, CHUNK), jnp.int32),
            pltpu.VMEM((NCH, CHUNK), jnp.int32),
            pltpu.VMEM((3, CHUNK, HD), jnp.float32),
            pltpu.VMEM((3, CHUNK, HD), jnp.float32),
            pltpu.VMEM((3, CHUNK, HD), jnp.float32),
            pltpu.VMEM((HD,), jnp.float32),
            pltpu.VMEM_SHARED((N, HD), jnp.float32),
            (pltpu.SemaphoreType.DMA,) * 3,
            (pltpu.SemaphoreType.DMA,) * 3,
            (pltpu.SemaphoreType.DMA,) * 3,
        ],
        compiler_params=pltpu.CompilerParams(use_tc_tiling_on_sc=False, disable_bounds_checks=True, disable_semaphore_checks=True),
    )
    def k(src_hbm, tgt_hbm, s1_hbm, s2_hbm, mt_hbm, z_hbm,
          ex_hbm, n0_hbm, n1_hbm,
          src_all, tgt_all, gs2, gt2, ex2, mt_v, norm_sh, semg, semw,
          semsc):
        cid = lax.axis_index("c")
        sid = lax.axis_index("s")
        wid = sid * NC + cid
        _node_slab_copy(sid, lambda o, n: z_hbm.at[pl.ds(0, n)],
                        lambda o, n: norm_sh.at[pl.ds(o, n)])
        pltpu.sync_copy(mt_hbm, mt_v)
        pltpu.sync_copy(src_hbm.at[wid], src_all)
        pltpu.sync_copy(tgt_hbm.at[wid], tgt_all)
        plsc.subcore_barrier()
        mt_row = mt_v[...]
        base = wid * EPT
        gs = [gs2.at[0], gs2.at[1], gs2.at[2]]
        gt = [gt2.at[0], gt2.at[1], gt2.at[2]]
        exv = [ex2.at[0], ex2.at[1], ex2.at[2]]

        def descs(slot, ci):
            return (
                pltpu.make_async_copy(s1_hbm.at[src_all.at[ci]], gs[slot],
                                      semg[slot]),
                pltpu.make_async_copy(s2_hbm.at[tgt_all.at[ci]], gt[slot],
                                      semg[slot]),
            )

        def wr_descs(slot, ci):
            off = pl.multiple_of(base + ci * CHUNK, 8)
            return (
                pltpu.make_async_copy(exv[slot], ex_hbm.at[pl.ds(off, CHUNK)],
                                      semw[slot]),
                pltpu.make_async_copy(exv[slot], norm_sh.at[tgt_all.at[ci]],
                                      semsc[slot]),
            )

        def start(slot, ci):
            d1, d2 = descs(slot, ci)
            d1.start()
            d2.start()

        def work(slot, ci):
            d1, d2 = descs(slot, ci)
            d1.wait()
            d2.wait()
            dw, dsc = wr_descs(slot, ci)

            @pl.when(jnp.asarray(ci, jnp.int32) >= 3)
            def _():
                dw.wait()
                dsc.wait()

            g_s = gs[slot]
            g_t = gt[slot]
            e_v = exv[slot]
            for i in range(CHUNK):
                s = g_s[i, :] + g_t[i, :]
                s = jnp.where(s > 0, s, ALPHA * s)
                e_v[i, :] = jnp.exp(s - mt_row)
            dw.start()
            dsc.start(add=True)

        start(0, 0)
        start(1, 1)
        ntrip = (NCH - 2) // 3

        def triple(i, carry):
            c = 3 * i
            for kk in range(3):
                work(kk, c + kk)
                start((kk + 2) % 3, c + kk + 2)
            return carry

        lax.fori_loop(0, ntrip, triple, 0)
        for c in range(3 * ntrip, NCH):
            work(c % 3, c)
            if c + 2 < NCH:
                start((c + 2) % 3, c + 2)
        for slot in range(3):
            dw, dsc = wr_descs(slot, 0)
            dw.wait()
            dsc.wait()
        plsc.subcore_barrier()

        @pl.when(cid == 0)
        def _():
            _node_slab_copy(sid, lambda o, n: norm_sh.at[pl.ds(o, n)],
                            lambda o, n: n0_hbm.at[pl.ds(o, n)])

        @pl.when(cid == 1)
        def _():
            _node_slab_copy(sid, lambda o, n: norm_sh.at[pl.ds(o, n)],
                            lambda o, n: n1_hbm.at[pl.ds(o, n)])

    return k(src_r, tgt_r, s1t, s2t, mt, z16)


def _sc_aggregate(src_r, tgt_r, ex, n0, n1, h, z128):
    mesh = plsc.VectorSubcoreMesh(core_axis_name="c", subcore_axis_name="s")

    @functools.partial(
        pl.kernel,
        out_type=(
            jax.ShapeDtypeStruct((N, F), jnp.float32),
            jax.ShapeDtypeStruct((N, F), jnp.float32),
            jax.ShapeDtypeStruct((N, HD), jnp.float32),
        ),
        mesh=mesh,
        scratch_types=[
            pltpu.VMEM((ANCH, ACHUNK), jnp.int32),
            pltpu.VMEM((ANCH, ACHUNK), jnp.int32),
            pltpu.VMEM((3, ACHUNK, F), jnp.float32),
            pltpu.VMEM((3, ACHUNK, HD), jnp.float32),
            pltpu.VMEM((3, ACHUNK, HD), jnp.float32),
            pltpu.VMEM((NB, HD), jnp.float32),
            pltpu.VMEM((NB, HD), jnp.float32),
            pltpu.VMEM_SHARED((N, F), jnp.float32),
            (pltpu.SemaphoreType.DMA,) * 3,
            (pltpu.SemaphoreType.DMA,) * 3,
            (pltpu.SemaphoreType.DMA,) * 3,
            (pltpu.SemaphoreType.DMA,) * 3,
        ],
        compiler_params=pltpu.CompilerParams(use_tc_tiling_on_sc=False, disable_bounds_checks=True, disable_semaphore_checks=True),
    )
    def k(src_hbm, tgt_hbm, ex_hbm, n0_hbm, n1_hbm, h_hbm, z_hbm,
          p0_hbm, p1_hbm, rn_hbm,
          src_all, tgt_all, hb3, g03, ex3, nb0, nb1, out_sh,
          semh, semn, seme, semsc):
        cid = lax.axis_index("c")
        sid = lax.axis_index("s")
        wid = sid * NC + cid
        _node_slab_copy(sid, lambda o, n: z_hbm.at[pl.ds(0, n)],
                        lambda o, n: out_sh.at[pl.ds(o, n)])
        pltpu.sync_copy(src_hbm.at[wid], src_all)
        pltpu.sync_copy(tgt_hbm.at[wid], tgt_all)

        def rn_block(off, nrows):
            pltpu.sync_copy(n0_hbm.at[pl.ds(off, nrows)],
                            nb0.at[pl.ds(0, nrows)])
            pltpu.sync_copy(n1_hbm.at[pl.ds(off, nrows)],
                            nb1.at[pl.ds(0, nrows)])
            for r in range(nrows):
                nb0[r, :] = 1.0 / (nb0[r, :] + nb1[r, :])
            pltpu.sync_copy(nb0.at[pl.ds(0, nrows)],
                            rn_hbm.at[pl.ds(off, nrows)])

        def rn_loop(b, carry):
            rn_block(pl.multiple_of(sid * RPS + b * NB, 8), NB)
            return carry

        lax.fori_loop(0, RPS // NB, rn_loop, 0)

        @pl.when(sid == NS - 1)
        def _():
            rn_block(15 * RPS + RPS // NB * NB, RPS_LAST - RPS)

        plsc.subcore_barrier()
        base = wid * EPT
        hb = [hb3.at[0], hb3.at[1], hb3.at[2]]
        g0 = [g03.at[0], g03.at[1], g03.at[2]]
        exv = [ex3.at[0], ex3.at[1], ex3.at[2]]

        def descs(slot, ci):
            off = pl.multiple_of(base + ci * ACHUNK, 8)
            return (
                pltpu.make_async_copy(h_hbm.at[src_all.at[ci]], hb[slot],
                                      semh[slot]),
                pltpu.make_async_copy(rn_hbm.at[tgt_all.at[ci]], g0[slot],
                                      semn[slot]),
                pltpu.make_async_copy(ex_hbm.at[pl.ds(off, ACHUNK)],
                                      exv[slot], seme[slot]),
            )

        def sc_desc(slot, ci):
            return pltpu.make_async_copy(hb[slot], out_sh.at[tgt_all.at[ci]],
                                         semsc[slot])

        def start(slot, ci):
            @pl.when(jnp.asarray(ci, jnp.int32) >= 3)
            def _():
                sc_desc(slot, ci).wait()
            for d in descs(slot, ci):
                d.start()

        def work(slot, ci):
            dh, d0, de = descs(slot, ci)
            d0.wait()
            de.wait()
            dh.wait()
            e_v = exv[slot]
            g_0 = g0[slot]
            h_b = hb[slot]
            for i in range(ACHUNK):
                row = e_v[i, :] * g_0[i, :]
                for j in range(H):
                    a_b = _lane_bcast(row, j)
                    h_b[i, pl.ds(j * HD, HD)] = (
                        h_b[i, pl.ds(j * HD, HD)] * a_b)
            sc_desc(slot, ci).start(add=True)

        start(0, 0)
        start(1, 1)
        ntrip = (ANCH - 2) // 3

        def triple(i, carry):
            c = 3 * i
            for kk in range(3):
                work(kk, c + kk)
                start((kk + 2) % 3, c + kk + 2)
            return carry

        lax.fori_loop(0, ntrip, triple, 0)
        for c in range(3 * ntrip, ANCH):
            work(c % 3, c)
            if c + 2 < ANCH:
                start((c + 2) % 3, c + 2)
        for slot in range(3):
            pltpu.make_async_copy(hb[slot], out_sh.at[tgt_all.at[0]],
                                  semsc[slot]).wait()
        plsc.subcore_barrier()

        @pl.when(cid == 0)
        def _():
            _node_slab_copy(sid, lambda o, n: out_sh.at[pl.ds(o, n)],
                            lambda o, n: p0_hbm.at[pl.ds(o, n)])

        @pl.when(cid == 1)
        def _():
            _node_slab_copy(sid, lambda o, n: out_sh.at[pl.ds(o, n)],
                            lambda o, n: p1_hbm.at[pl.ds(o, n)])

    return k(src_r, tgt_r, ex, n0, n1, h, z128)


def _tc_combine_body(p0_ref, p1_ref, b_ref, o_ref):
    o_ref[...] = p0_ref[...] + p1_ref[...] + b_ref[...]


def _tc_combine(p0, p1, bias, interpret=False):
    return pl.pallas_call(
        _tc_combine_body,
        grid=(N // BN,),
        in_specs=[
            pl.BlockSpec((BN, F), lambda i: (i, 0)),
            pl.BlockSpec((BN, F), lambda i: (i, 0)),
            pl.BlockSpec((1, F), lambda i: (0, 0)),
        ],
        out_specs=pl.BlockSpec((BN, F), lambda i: (i, 0)),
        out_shape=jax.ShapeDtypeStruct((N, F), jnp.float32),
        interpret=interpret,
    )(p0, p1, bias.reshape(1, F))


def kernel(x, edge_index, W, a, bias):
    src = edge_index[0].reshape(NW, NCH, CHUNK)
    tgt = edge_index[1].reshape(NW, NCH, CHUNK)
    src_a = edge_index[0].reshape(NW, ANCH, ACHUNK)
    tgt_a = edge_index[1].reshape(NW, ANCH, ACHUNK)
    wall = jnp.transpose(W, (1, 0, 2)).reshape(F, F)
    eye8 = jnp.eye(H, dtype=jnp.float32)
    a1 = a[:, :HD, 0]
    a2 = a[:, HD:, 0]
    A1 = jnp.einsum('gk,hg->hkg', a1, eye8).reshape(F, H)
    A2 = jnp.einsum('gk,hg->hkg', a2, eye8).reshape(F, H)
    a1t = jnp.concatenate([A1, A1], axis=1)
    a2t = jnp.concatenate([A2, A2], axis=1)

    h, s1t, s2t = _tc_head(x, wall, a1t, a2t)
    mt = jnp.maximum(jnp.max(s1t, axis=0) + jnp.max(s2t, axis=0), 0.0)

    z16 = jnp.zeros((RPS_LAST, HD), jnp.float32)
    ex, n0, n1 = _sc_scores(src, tgt, s1t, s2t, mt, z16)

    z128 = jnp.zeros((RPS_LAST, F), jnp.float32)
    p0, p1, _ = _sc_aggregate(src_a, tgt_a, ex, n0, n1, h, z128)

    return _tc_combine(p0, p1, bias)


# speedup vs baseline: 1.0597x; 1.0581x over previous
import functools

import jax
import jax.numpy as jnp
from jax import lax
from jax.experimental import pallas as pl
from jax.experimental.pallas import tpu as pltpu
from jax.experimental.pallas import tpu_sc as plsc

N = 10000
E = 320000
F = 128
H = 8
HD = 16
ALPHA = 0.2

NC = 2
NS = 16
NW = NC * NS
EPT = E // NW
CHUNK = 80
NCH = EPT // CHUNK
ACHUNK = 40
ANCH = EPT // ACHUNK
NB = 104
RPS = 624
RPS_LAST = N - 15 * RPS


def _node_slab_copy(sid, src_fn, dst_fn):
    off = pl.multiple_of(sid * RPS, 8)

    @pl.when(sid < NS - 1)
    def _():
        pltpu.sync_copy(src_fn(off, RPS), dst_fn(off, RPS))

    @pl.when(sid == NS - 1)
    def _():
        pltpu.sync_copy(src_fn(15 * RPS, RPS_LAST), dst_fn(15 * RPS, RPS_LAST))

BN = 2000

_GDN = lax.GatherDimensionNumbers(
    offset_dims=(), collapsed_slice_dims=(0,), start_index_map=(0,))


def _lane_bcast(row, j):
    idx = jnp.full((HD, 1), j, jnp.int32)
    return lax.gather(row, idx, _GDN, (1,),
                      mode=lax.GatherScatterMode.PROMISE_IN_BOUNDS)


def _tc_head_body(x_ref, wall_ref, a1_ref, a2_ref, h_ref, s1_ref, s2_ref):
    h = jnp.dot(x_ref[...], wall_ref[...], preferred_element_type=jnp.float32)
    h_ref[...] = h
    s1_ref[...] = jnp.dot(h, a1_ref[...], preferred_element_type=jnp.float32)
    s2_ref[...] = jnp.dot(h, a2_ref[...], preferred_element_type=jnp.float32)


def _tc_head(x, wall, a1t, a2t, interpret=False):
    return pl.pallas_call(
        _tc_head_body,
        grid=(N // BN,),
        in_specs=[
            pl.BlockSpec((BN, F), lambda i: (i, 0)),
            pl.BlockSpec((F, F), lambda i: (0, 0)),
            pl.BlockSpec((F, HD), lambda i: (0, 0)),
            pl.BlockSpec((F, HD), lambda i: (0, 0)),
        ],
        out_specs=[
            pl.BlockSpec((BN, F), lambda i: (i, 0)),
            pl.BlockSpec((BN, HD), lambda i: (i, 0)),
            pl.BlockSpec((BN, HD), lambda i: (i, 0)),
        ],
        out_shape=[
            jax.ShapeDtypeStruct((N, F), jnp.float32),
            jax.ShapeDtypeStruct((N, HD), jnp.float32),
            jax.ShapeDtypeStruct((N, HD), jnp.float32),
        ],
        interpret=interpret,
    )(x, wall, a1t, a2t)


def _sc_scores(src_r, tgt_r, s1t, s2t, mt, z16):
    mesh = plsc.VectorSubcoreMesh(core_axis_name="c", subcore_axis_name="s")

    @functools.partial(
        pl.kernel,
        out_type=(
            jax.ShapeDtypeStruct((E, HD), jnp.float32),
            jax.ShapeDtypeStruct((N, HD), jnp.float32),
            jax.ShapeDtypeStruct((N, HD), jnp.float32),
        ),
        mesh=mesh,
        scratch_types=[
            pltpu.VMEM((NCH, CHUNK), jnp.int32),
            pltpu.VMEM((NCH, CHUNK), jnp.int32),
            pltpu.VMEM((3, CHUNK, HD), jnp.float32),
            pltpu.VMEM((3, CHUNK, HD), jnp.float32),
            pltpu.VMEM((3, CHUNK, HD), jnp.float32),
            pltpu.VMEM((HD,), jnp.float32),
            pltpu.VMEM_SHARED((N, HD), jnp.float32),
            (pltpu.SemaphoreType.DMA,) * 3,
            (pltpu.SemaphoreType.DMA,) * 3,
            (pltpu.SemaphoreType.DMA,) * 3,
        ],
        compiler_params=pltpu.CompilerParams(use_tc_tiling_on_sc=False),
    )
    def k(src_hbm, tgt_hbm, s1_hbm, s2_hbm, mt_hbm, z_hbm,
          ex_hbm, n0_hbm, n1_hbm,
          src_all, tgt_all, gs2, gt2, ex2, mt_v, norm_sh, semg, semw,
          semsc):
        cid = lax.axis_index("c")
        sid = lax.axis_index("s")
        wid = sid * NC + cid
        _node_slab_copy(sid, lambda o, n: z_hbm.at[pl.ds(0, n)],
                        lambda o, n: norm_sh.at[pl.ds(o, n)])
        pltpu.sync_copy(mt_hbm, mt_v)
        pltpu.sync_copy(src_hbm.at[wid], src_all)
        pltpu.sync_copy(tgt_hbm.at[wid], tgt_all)
        plsc.subcore_barrier()
        mt_row = mt_v[...]
        base = wid * EPT
        gs = [gs2.at[0], gs2.at[1], gs2.at[2]]
        gt = [gt2.at[0], gt2.at[1], gt2.at[2]]
        exv = [ex2.at[0], ex2.at[1], ex2.at[2]]

        def descs(slot, ci):
            return (
                pltpu.make_async_copy(s1_hbm.at[src_all.at[ci]], gs[slot],
                                      semg[slot]),
                pltpu.make_async_copy(s2_hbm.at[tgt_all.at[ci]], gt[slot],
                                      semg[slot]),
            )

        def wr_descs(slot, ci):
            off = pl.multiple_of(base + ci * CHUNK, 8)
            return (
                pltpu.make_async_copy(exv[slot], ex_hbm.at[pl.ds(off, CHUNK)],
                                      semw[slot]),
                pltpu.make_async_copy(exv[slot], norm_sh.at[tgt_all.at[ci]],
                                      semsc[slot]),
            )

        def start(slot, ci):
            d1, d2 = descs(slot, ci)
            d1.start()
            d2.start()

        def work(slot, ci):
            d1, d2 = descs(slot, ci)
            d1.wait()
            d2.wait()
            dw, dsc = wr_descs(slot, ci)

            @pl.when(jnp.asarray(ci, jnp.int32) >= 3)
            def _():
                dw.wait()
                dsc.wait()

            g_s = gs[slot]
            g_t = gt[slot]
            e_v = exv[slot]
            for i in range(CHUNK):
                s = g_s[i, :] + g_t[i, :]
                s = jnp.where(s > 0, s, ALPHA * s)
                e_v[i, :] = jnp.exp(s - mt_row)
            dw.start()
            dsc.start(add=True)

        start(0, 0)
        start(1, 1)
        ntrip = (NCH - 2) // 3

        def triple(i, carry):
            c = 3 * i
            for kk in range(3):
                work(kk, c + kk)
                start((kk + 2) % 3, c + kk + 2)
            return carry

        lax.fori_loop(0, ntrip, triple, 0)
        for c in range(3 * ntrip, NCH):
            work(c % 3, c)
            if c + 2 < NCH:
                start((c + 2) % 3, c + 2)
        for slot in range(3):
            dw, dsc = wr_descs(slot, 0)
            dw.wait()
            dsc.wait()
        plsc.subcore_barrier()

        @pl.when(cid == 0)
        def _():
            _node_slab_copy(sid, lambda o, n: norm_sh.at[pl.ds(o, n)],
                            lambda o, n: n0_hbm.at[pl.ds(o, n)])

        @pl.when(cid == 1)
        def _():
            _node_slab_copy(sid, lambda o, n: norm_sh.at[pl.ds(o, n)],
                            lambda o, n: n1_hbm.at[pl.ds(o, n)])

    return k(src_r, tgt_r, s1t, s2t, mt, z16)


def _sc_aggregate(src_r, tgt_r, ex, n0, n1, h, z128):
    mesh = plsc.VectorSubcoreMesh(core_axis_name="c", subcore_axis_name="s")

    @functools.partial(
        pl.kernel,
        out_type=(
            jax.ShapeDtypeStruct((N, F), jnp.float32),
            jax.ShapeDtypeStruct((N, F), jnp.float32),
            jax.ShapeDtypeStruct((N, HD), jnp.float32),
        ),
        mesh=mesh,
        scratch_types=[
            pltpu.VMEM((ANCH, ACHUNK), jnp.int32),
            pltpu.VMEM((ANCH, ACHUNK), jnp.int32),
            pltpu.VMEM((3, ACHUNK, F), jnp.float32),
            pltpu.VMEM((3, ACHUNK, HD), jnp.float32),
            pltpu.VMEM((3, ACHUNK, HD), jnp.float32),
            pltpu.VMEM((NB, HD), jnp.float32),
            pltpu.VMEM((NB, HD), jnp.float32),
            pltpu.VMEM_SHARED((N, F), jnp.float32),
            (pltpu.SemaphoreType.DMA,) * 3,
            (pltpu.SemaphoreType.DMA,) * 3,
            (pltpu.SemaphoreType.DMA,) * 3,
            (pltpu.SemaphoreType.DMA,) * 3,
        ],
        compiler_params=pltpu.CompilerParams(use_tc_tiling_on_sc=False),
    )
    def k(src_hbm, tgt_hbm, ex_hbm, n0_hbm, n1_hbm, h_hbm, z_hbm,
          p0_hbm, p1_hbm, rn_hbm,
          src_all, tgt_all, hb3, g03, ex3, nb0, nb1, out_sh,
          semh, semn, seme, semsc):
        cid = lax.axis_index("c")
        sid = lax.axis_index("s")
        wid = sid * NC + cid
        _node_slab_copy(sid, lambda o, n: z_hbm.at[pl.ds(0, n)],
                        lambda o, n: out_sh.at[pl.ds(o, n)])
        pltpu.sync_copy(src_hbm.at[wid], src_all)
        pltpu.sync_copy(tgt_hbm.at[wid], tgt_all)

        def rn_block(off, nrows):
            pltpu.sync_copy(n0_hbm.at[pl.ds(off, nrows)],
                            nb0.at[pl.ds(0, nrows)])
            pltpu.sync_copy(n1_hbm.at[pl.ds(off, nrows)],
                            nb1.at[pl.ds(0, nrows)])
            for r in range(nrows):
                nb0[r, :] = 1.0 / (nb0[r, :] + nb1[r, :])
            pltpu.sync_copy(nb0.at[pl.ds(0, nrows)],
                            rn_hbm.at[pl.ds(off, nrows)])

        def rn_loop(b, carry):
            rn_block(pl.multiple_of(sid * RPS + b * NB, 8), NB)
            return carry

        lax.fori_loop(0, RPS // NB, rn_loop, 0)

        @pl.when(sid == NS - 1)
        def _():
            rn_block(15 * RPS + RPS // NB * NB, RPS_LAST - RPS)

        plsc.subcore_barrier()
        base = wid * EPT
        hb = [hb3.at[0], hb3.at[1], hb3.at[2]]
        g0 = [g03.at[0], g03.at[1], g03.at[2]]
        exv = [ex3.at[0], ex3.at[1], ex3.at[2]]

        def descs(slot, ci):
            off = pl.multiple_of(base + ci * ACHUNK, 8)
            return (
                pltpu.make_async_copy(h_hbm.at[src_all.at[ci]], hb[slot],
                                      semh[slot]),
                pltpu.make_async_copy(rn_hbm.at[tgt_all.at[ci]], g0[slot],
                                      semn[slot]),
                pltpu.make_async_copy(ex_hbm.at[pl.ds(off, ACHUNK)],
                                      exv[slot], seme[slot]),
            )

        def sc_desc(slot, ci):
            return pltpu.make_async_copy(hb[slot], out_sh.at[tgt_all.at[ci]],
                                         semsc[slot])

        def start(slot, ci):
            @pl.when(jnp.asarray(ci, jnp.int32) >= 3)
            def _():
                sc_desc(slot, ci).wait()
            for d in descs(slot, ci):
                d.start()

        def work(slot, ci):
            dh, d0, de = descs(slot, ci)
            d0.wait()
            de.wait()
            dh.wait()
            e_v = exv[slot]
            g_0 = g0[slot]
            h_b = hb[slot]
            for i in range(ACHUNK):
                row = e_v[i, :] * g_0[i, :]
                for j in range(H):
                    a_b = _lane_bcast(row, j)
                    h_b[i, pl.ds(j * HD, HD)] = (
                        h_b[i, pl.ds(j * HD, HD)] * a_b)
            sc_desc(slot, ci).start(add=True)

        start(0, 0)
        start(1, 1)
        ntrip = (ANCH - 2) // 3

        def triple(i, carry):
            c = 3 * i
            for kk in range(3):
                work(kk, c + kk)
                start((kk + 2) % 3, c + kk + 2)
            return carry

        lax.fori_loop(0, ntrip, triple, 0)
        for c in range(3 * ntrip, ANCH):
            work(c % 3, c)
            if c + 2 < ANCH:
                start((c + 2) % 3, c + 2)
        for slot in range(3):
            pltpu.make_async_copy(hb[slot], out_sh.at[tgt_all.at[0]],
                                  semsc[slot]).wait()
        plsc.subcore_barrier()

        @pl.when(cid == 0)
        def _():
            _node_slab_copy(sid, lambda o, n: out_sh.at[pl.ds(o, n)],
                            lambda o, n: p0_hbm.at[pl.ds(o, n)])

        @pl.when(cid == 1)
        def _():
            _node_slab_copy(sid, lambda o, n: out_sh.at[pl.ds(o, n)],
                            lambda o, n: p1_hbm.at[pl.ds(o, n)])

    return k(src_r, tgt_r, ex, n0, n1, h, z128)


def _tc_combine_body(p0_ref, p1_ref, b_ref, o_ref):
    o_ref[...] = p0_ref[...] + p1_ref[...] + b_ref[...]


def _tc_combine(p0, p1, bias, interpret=False):
    return pl.pallas_call(
        _tc_combine_body,
        grid=(N // BN,),
        in_specs=[
            pl.BlockSpec((BN, F), lambda i: (i, 0)),
            pl.BlockSpec((BN, F), lambda i: (i, 0)),
            pl.BlockSpec((1, F), lambda i: (0, 0)),
        ],
        out_specs=pl.BlockSpec((BN, F), lambda i: (i, 0)),
        out_shape=jax.ShapeDtypeStruct((N, F), jnp.float32),
        interpret=interpret,
    )(p0, p1, bias.reshape(1, F))


def kernel(x, edge_index, W, a, bias):
    src = edge_index[0].reshape(NW, NCH, CHUNK)
    tgt = edge_index[1].reshape(NW, NCH, CHUNK)
    src_a = edge_index[0].reshape(NW, ANCH, ACHUNK)
    tgt_a = edge_index[1].reshape(NW, ANCH, ACHUNK)
    wall = jnp.transpose(W, (1, 0, 2)).reshape(F, F)
    eye8 = jnp.eye(H, dtype=jnp.float32)
    a1 = a[:, :HD, 0]
    a2 = a[:, HD:, 0]
    A1 = jnp.einsum('gk,hg->hkg', a1, eye8).reshape(F, H)
    A2 = jnp.einsum('gk,hg->hkg', a2, eye8).reshape(F, H)
    a1t = jnp.concatenate([A1, A1], axis=1)
    a2t = jnp.concatenate([A2, A2], axis=1)

    h, s1t, s2t = _tc_head(x, wall, a1t, a2t)
    mt = jnp.maximum(jnp.max(s1t, axis=0) + jnp.max(s2t, axis=0), 0.0)

    z16 = jnp.zeros((RPS_LAST, HD), jnp.float32)
    ex, n0, n1 = _sc_scores(src, tgt, s1t, s2t, mt, z16)

    z128 = jnp.zeros((RPS_LAST, F), jnp.float32)
    p0, p1, _ = _sc_aggregate(src_a, tgt_a, ex, n0, n1, h, z128)

    return _tc_combine(p0, p1, bias)


# speedup vs baseline: 1.0646x; 1.0046x over previous
import functools

import jax
import jax.numpy as jnp
from jax import lax
from jax.experimental import pallas as pl
from jax.experimental.pallas import tpu as pltpu
from jax.experimental.pallas import tpu_sc as plsc

N = 10000
E = 320000
F = 128
H = 8
HD = 16
ALPHA = 0.2

NC = 2
NS = 16
NW = NC * NS
EPT = E // NW
CHUNK = 80
NCH = EPT // CHUNK
ACHUNK = 40
ANCH = EPT // ACHUNK
NB = 104
RPS = 624
RPS_LAST = N - 15 * RPS


def _node_slab_copy(sid, src_fn, dst_fn):
    off = pl.multiple_of(sid * RPS, 8)

    @pl.when(sid < NS - 1)
    def _():
        pltpu.sync_copy(src_fn(off, RPS), dst_fn(off, RPS))

    @pl.when(sid == NS - 1)
    def _():
        pltpu.sync_copy(src_fn(15 * RPS, RPS_LAST), dst_fn(15 * RPS, RPS_LAST))

BN = 10000

_GDN = lax.GatherDimensionNumbers(
    offset_dims=(), collapsed_slice_dims=(0,), start_index_map=(0,))


def _lane_bcast(row, j):
    idx = jnp.full((HD, 1), j, jnp.int32)
    return lax.gather(row, idx, _GDN, (1,),
                      mode=lax.GatherScatterMode.PROMISE_IN_BOUNDS)


def _tc_head_body(x_ref, wall_ref, a1_ref, a2_ref, h_ref, s1_ref, s2_ref):
    h = jnp.dot(x_ref[...], wall_ref[...], preferred_element_type=jnp.float32)
    h_ref[...] = h
    s1_ref[...] = jnp.dot(h, a1_ref[...], preferred_element_type=jnp.float32)
    s2_ref[...] = jnp.dot(h, a2_ref[...], preferred_element_type=jnp.float32)


def _tc_head(x, wall, a1t, a2t, interpret=False):
    return pl.pallas_call(
        _tc_head_body,
        grid=(N // BN,),
        in_specs=[
            pl.BlockSpec((BN, F), lambda i: (i, 0)),
            pl.BlockSpec((F, F), lambda i: (0, 0)),
            pl.BlockSpec((F, HD), lambda i: (0, 0)),
            pl.BlockSpec((F, HD), lambda i: (0, 0)),
        ],
        out_specs=[
            pl.BlockSpec((BN, F), lambda i: (i, 0)),
            pl.BlockSpec((BN, HD), lambda i: (i, 0)),
            pl.BlockSpec((BN, HD), lambda i: (i, 0)),
        ],
        out_shape=[
            jax.ShapeDtypeStruct((N, F), jnp.float32),
            jax.ShapeDtypeStruct((N, HD), jnp.float32),
            jax.ShapeDtypeStruct((N, HD), jnp.float32),
        ],
        interpret=interpret,
    )(x, wall, a1t, a2t)


def _sc_scores(src_r, tgt_r, s1t, s2t, mt, z16):
    mesh = plsc.VectorSubcoreMesh(core_axis_name="c", subcore_axis_name="s")

    @functools.partial(
        pl.kernel,
        out_type=(
            jax.ShapeDtypeStruct((E, HD), jnp.float32),
            jax.ShapeDtypeStruct((N, HD), jnp.float32),
            jax.ShapeDtypeStruct((N, HD), jnp.float32),
        ),
        mesh=mesh,
        scratch_types=[
            pltpu.VMEM((NCH, CHUNK), jnp.int32),
            pltpu.VMEM((NCH, CHUNK), jnp.int32),
            pltpu.VMEM((3, CHUNK, HD), jnp.float32),
            pltpu.VMEM((3, CHUNK, HD), jnp.float32),
            pltpu.VMEM((3, CHUNK, HD), jnp.float32),
            pltpu.VMEM((HD,), jnp.float32),
            pltpu.VMEM_SHARED((N, HD), jnp.float32),
            (pltpu.SemaphoreType.DMA,) * 3,
            (pltpu.SemaphoreType.DMA,) * 3,
            (pltpu.SemaphoreType.DMA,) * 3,
        ],
        compiler_params=pltpu.CompilerParams(use_tc_tiling_on_sc=False),
    )
    def k(src_hbm, tgt_hbm, s1_hbm, s2_hbm, mt_hbm, z_hbm,
          ex_hbm, n0_hbm, n1_hbm,
          src_all, tgt_all, gs2, gt2, ex2, mt_v, norm_sh, semg, semw,
          semsc):
        cid = lax.axis_index("c")
        sid = lax.axis_index("s")
        wid = sid * NC + cid
        _node_slab_copy(sid, lambda o, n: z_hbm.at[pl.ds(0, n)],
                        lambda o, n: norm_sh.at[pl.ds(o, n)])
        pltpu.sync_copy(mt_hbm, mt_v)
        pltpu.sync_copy(src_hbm.at[wid], src_all)
        pltpu.sync_copy(tgt_hbm.at[wid], tgt_all)
        plsc.subcore_barrier()
        mt_row = mt_v[...]
        base = wid * EPT
        gs = [gs2.at[0], gs2.at[1], gs2.at[2]]
        gt = [gt2.at[0], gt2.at[1], gt2.at[2]]
        exv = [ex2.at[0], ex2.at[1], ex2.at[2]]

        def descs(slot, ci):
            return (
                pltpu.make_async_copy(s1_hbm.at[src_all.at[ci]], gs[slot],
                                      semg[slot]),
                pltpu.make_async_copy(s2_hbm.at[tgt_all.at[ci]], gt[slot],
                                      semg[slot]),
            )

        def wr_descs(slot, ci):
            off = pl.multiple_of(base + ci * CHUNK, 8)
            return (
                pltpu.make_async_copy(exv[slot], ex_hbm.at[pl.ds(off, CHUNK)],
                                      semw[slot]),
                pltpu.make_async_copy(exv[slot], norm_sh.at[tgt_all.at[ci]],
                                      semsc[slot]),
            )

        def start(slot, ci):
            d1, d2 = descs(slot, ci)
            d1.start()
            d2.start()

        def work(slot, ci):
            d1, d2 = descs(slot, ci)
            d1.wait()
            d2.wait()
            dw, dsc = wr_descs(slot, ci)

            @pl.when(jnp.asarray(ci, jnp.int32) >= 3)
            def _():
                dw.wait()
                dsc.wait()

            g_s = gs[slot]
            g_t = gt[slot]
            e_v = exv[slot]
            for i in range(CHUNK):
                s = g_s[i, :] + g_t[i, :]
                s = jnp.where(s > 0, s, ALPHA * s)
                e_v[i, :] = jnp.exp(s - mt_row)
            dw.start()
            dsc.start(add=True)

        start(0, 0)
        start(1, 1)
        ntrip = (NCH - 2) // 3

        def triple(i, carry):
            c = 3 * i
            for kk in range(3):
                work(kk, c + kk)
                start((kk + 2) % 3, c + kk + 2)
            return carry

        lax.fori_loop(0, ntrip, triple, 0)
        for c in range(3 * ntrip, NCH):
            work(c % 3, c)
            if c + 2 < NCH:
                start((c + 2) % 3, c + 2)
        for slot in range(3):
            dw, dsc = wr_descs(slot, 0)
            dw.wait()
            dsc.wait()
        plsc.subcore_barrier()

        @pl.when(cid == 0)
        def _():
            _node_slab_copy(sid, lambda o, n: norm_sh.at[pl.ds(o, n)],
                            lambda o, n: n0_hbm.at[pl.ds(o, n)])

        @pl.when(cid == 1)
        def _():
            _node_slab_copy(sid, lambda o, n: norm_sh.at[pl.ds(o, n)],
                            lambda o, n: n1_hbm.at[pl.ds(o, n)])

    return k(src_r, tgt_r, s1t, s2t, mt, z16)


def _sc_aggregate(src_r, tgt_r, ex, n0, n1, h, z128):
    mesh = plsc.VectorSubcoreMesh(core_axis_name="c", subcore_axis_name="s")

    @functools.partial(
        pl.kernel,
        out_type=(
            jax.ShapeDtypeStruct((N, F), jnp.float32),
            jax.ShapeDtypeStruct((N, F), jnp.float32),
            jax.ShapeDtypeStruct((N, HD), jnp.float32),
        ),
        mesh=mesh,
        scratch_types=[
            pltpu.VMEM((ANCH, ACHUNK), jnp.int32),
            pltpu.VMEM((ANCH, ACHUNK), jnp.int32),
            pltpu.VMEM((3, ACHUNK, F), jnp.float32),
            pltpu.VMEM((3, ACHUNK, HD), jnp.float32),
            pltpu.VMEM((3, ACHUNK, HD), jnp.float32),
            pltpu.VMEM((NB, HD), jnp.float32),
            pltpu.VMEM((NB, HD), jnp.float32),
            pltpu.VMEM_SHARED((N, F), jnp.float32),
            (pltpu.SemaphoreType.DMA,) * 3,
            (pltpu.SemaphoreType.DMA,) * 3,
            (pltpu.SemaphoreType.DMA,) * 3,
            (pltpu.SemaphoreType.DMA,) * 3,
        ],
        compiler_params=pltpu.CompilerParams(use_tc_tiling_on_sc=False),
    )
    def k(src_hbm, tgt_hbm, ex_hbm, n0_hbm, n1_hbm, h_hbm, z_hbm,
          p0_hbm, p1_hbm, rn_hbm,
          src_all, tgt_all, hb3, g03, ex3, nb0, nb1, out_sh,
          semh, semn, seme, semsc):
        cid = lax.axis_index("c")
        sid = lax.axis_index("s")
        wid = sid * NC + cid
        _node_slab_copy(sid, lambda o, n: z_hbm.at[pl.ds(0, n)],
                        lambda o, n: out_sh.at[pl.ds(o, n)])
        pltpu.sync_copy(src_hbm.at[wid], src_all)
        pltpu.sync_copy(tgt_hbm.at[wid], tgt_all)

        def rn_block(off, nrows):
            pltpu.sync_copy(n0_hbm.at[pl.ds(off, nrows)],
                            nb0.at[pl.ds(0, nrows)])
            pltpu.sync_copy(n1_hbm.at[pl.ds(off, nrows)],
                            nb1.at[pl.ds(0, nrows)])
            for r in range(nrows):
                nb0[r, :] = 1.0 / (nb0[r, :] + nb1[r, :])
            pltpu.sync_copy(nb0.at[pl.ds(0, nrows)],
                            rn_hbm.at[pl.ds(off, nrows)])

        def rn_loop(b, carry):
            rn_block(pl.multiple_of(sid * RPS + b * NB, 8), NB)
            return carry

        lax.fori_loop(0, RPS // NB, rn_loop, 0)

        @pl.when(sid == NS - 1)
        def _():
            rn_block(15 * RPS + RPS // NB * NB, RPS_LAST - RPS)

        plsc.subcore_barrier()
        base = wid * EPT
        hb = [hb3.at[0], hb3.at[1], hb3.at[2]]
        g0 = [g03.at[0], g03.at[1], g03.at[2]]
        exv = [ex3.at[0], ex3.at[1], ex3.at[2]]

        def descs(slot, ci):
            off = pl.multiple_of(base + ci * ACHUNK, 8)
            return (
                pltpu.make_async_copy(h_hbm.at[src_all.at[ci]], hb[slot],
                                      semh[slot]),
                pltpu.make_async_copy(rn_hbm.at[tgt_all.at[ci]], g0[slot],
                                      semn[slot]),
                pltpu.make_async_copy(ex_hbm.at[pl.ds(off, ACHUNK)],
                                      exv[slot], seme[slot]),
            )

        def sc_desc(slot, ci):
            return pltpu.make_async_copy(hb[slot], out_sh.at[tgt_all.at[ci]],
                                         semsc[slot])

        def start(slot, ci):
            @pl.when(jnp.asarray(ci, jnp.int32) >= 3)
            def _():
                sc_desc(slot, ci).wait()
            for d in descs(slot, ci):
                d.start()

        def work(slot, ci):
            dh, d0, de = descs(slot, ci)
            d0.wait()
            de.wait()
            dh.wait()
            e_v = exv[slot]
            g_0 = g0[slot]
            h_b = hb[slot]
            for i in range(ACHUNK):
                row = e_v[i, :] * g_0[i, :]
                for j in range(H):
                    a_b = _lane_bcast(row, j)
                    h_b[i, pl.ds(j * HD, HD)] = (
                        h_b[i, pl.ds(j * HD, HD)] * a_b)
            sc_desc(slot, ci).start(add=True)

        start(0, 0)
        start(1, 1)
        ntrip = (ANCH - 2) // 3

        def triple(i, carry):
            c = 3 * i
            for kk in range(3):
                work(kk, c + kk)
                start((kk + 2) % 3, c + kk + 2)
            return carry

        lax.fori_loop(0, ntrip, triple, 0)
        for c in range(3 * ntrip, ANCH):
            work(c % 3, c)
            if c + 2 < ANCH:
                start((c + 2) % 3, c + 2)
        for slot in range(3):
            pltpu.make_async_copy(hb[slot], out_sh.at[tgt_all.at[0]],
                                  semsc[slot]).wait()
        plsc.subcore_barrier()

        @pl.when(cid == 0)
        def _():
            _node_slab_copy(sid, lambda o, n: out_sh.at[pl.ds(o, n)],
                            lambda o, n: p0_hbm.at[pl.ds(o, n)])

        @pl.when(cid == 1)
        def _():
            _node_slab_copy(sid, lambda o, n: out_sh.at[pl.ds(o, n)],
                            lambda o, n: p1_hbm.at[pl.ds(o, n)])

    return k(src_r, tgt_r, ex, n0, n1, h, z128)


def _tc_combine_body(p0_ref, p1_ref, b_ref, o_ref):
    o_ref[...] = p0_ref[...] + p1_ref[...] + b_ref[...]


def _tc_combine(p0, p1, bias, interpret=False):
    return pl.pallas_call(
        _tc_combine_body,
        grid=(N // BN,),
        in_specs=[
            pl.BlockSpec((BN, F), lambda i: (i, 0)),
            pl.BlockSpec((BN, F), lambda i: (i, 0)),
            pl.BlockSpec((1, F), lambda i: (0, 0)),
        ],
        out_specs=pl.BlockSpec((BN, F), lambda i: (i, 0)),
        out_shape=jax.ShapeDtypeStruct((N, F), jnp.float32),
        interpret=interpret,
    )(p0, p1, bias.reshape(1, F))


def kernel(x, edge_index, W, a, bias):
    src = edge_index[0].reshape(NW, NCH, CHUNK)
    tgt = edge_index[1].reshape(NW, NCH, CHUNK)
    src_a = edge_index[0].reshape(NW, ANCH, ACHUNK)
    tgt_a = edge_index[1].reshape(NW, ANCH, ACHUNK)
    wall = jnp.transpose(W, (1, 0, 2)).reshape(F, F)
    eye8 = jnp.eye(H, dtype=jnp.float32)
    a1 = a[:, :HD, 0]
    a2 = a[:, HD:, 0]
    A1 = jnp.einsum('gk,hg->hkg', a1, eye8).reshape(F, H)
    A2 = jnp.einsum('gk,hg->hkg', a2, eye8).reshape(F, H)
    a1t = jnp.concatenate([A1, A1], axis=1)
    a2t = jnp.concatenate([A2, A2], axis=1)

    h, s1t, s2t = _tc_head(x, wall, a1t, a2t)
    mt = jnp.maximum(jnp.max(s1t, axis=0) + jnp.max(s2t, axis=0), 0.0)

    z16 = jnp.zeros((RPS_LAST, HD), jnp.float32)
    ex, n0, n1 = _sc_scores(src, tgt, s1t, s2t, mt, z16)

    z128 = jnp.zeros((RPS_LAST, F), jnp.float32)
    p0, p1, _ = _sc_aggregate(src_a, tgt_a, ex, n0, n1, h, z128)

    return _tc_combine(p0, p1, bias)


# speedup vs baseline: 1.0818x; 1.0161x over previous
import functools

import jax
import jax.numpy as jnp
from jax import lax
from jax.experimental import pallas as pl
from jax.experimental.pallas import tpu as pltpu
from jax.experimental.pallas import tpu_sc as plsc

N = 10000
E = 320000
F = 128
H = 8
HD = 16
ALPHA = 0.2

NC = 2
NS = 16
NW = NC * NS
EPT = E // NW
CHUNK = 80
NCH = EPT // CHUNK
ACHUNK = 40
ANCH = EPT // ACHUNK
NB = 104
RPS = 624
RPS_LAST = N - 15 * RPS


def _node_slab_copy(sid, src_fn, dst_fn):
    off = pl.multiple_of(sid * RPS, 8)

    @pl.when(sid < NS - 1)
    def _():
        pltpu.sync_copy(src_fn(off, RPS), dst_fn(off, RPS))

    @pl.when(sid == NS - 1)
    def _():
        pltpu.sync_copy(src_fn(15 * RPS, RPS_LAST), dst_fn(15 * RPS, RPS_LAST))

BN = 10000

_GDN = lax.GatherDimensionNumbers(
    offset_dims=(), collapsed_slice_dims=(0,), start_index_map=(0,))


def _lane_bcast(row, j):
    idx = jnp.full((HD, 1), j, jnp.int32)
    return lax.gather(row, idx, _GDN, (1,),
                      mode=lax.GatherScatterMode.PROMISE_IN_BOUNDS)


def _tc_head_body(x_ref, wall_ref, a1_ref, a2_ref, h_ref, s1_ref, s2_ref,
                  mt_ref):
    h = jnp.dot(x_ref[...], wall_ref[...], preferred_element_type=jnp.float32)
    h_ref[...] = h
    s1 = jnp.dot(h, a1_ref[...], preferred_element_type=jnp.float32)
    s2 = jnp.dot(h, a2_ref[...], preferred_element_type=jnp.float32)
    s1_ref[...] = s1
    s2_ref[...] = s2
    mt_ref[...] = jnp.maximum(
        jnp.max(s1, axis=0, keepdims=True) +
        jnp.max(s2, axis=0, keepdims=True), 0.0)


def _tc_head(x, wall, a1t, a2t, interpret=False):
    return pl.pallas_call(
        _tc_head_body,
        grid=(N // BN,),
        in_specs=[
            pl.BlockSpec((BN, F), lambda i: (i, 0)),
            pl.BlockSpec((F, F), lambda i: (0, 0)),
            pl.BlockSpec((F, HD), lambda i: (0, 0)),
            pl.BlockSpec((F, HD), lambda i: (0, 0)),
        ],
        out_specs=[
            pl.BlockSpec((BN, F), lambda i: (i, 0)),
            pl.BlockSpec((BN, HD), lambda i: (i, 0)),
            pl.BlockSpec((BN, HD), lambda i: (i, 0)),
            pl.BlockSpec((1, HD), lambda i: (0, 0)),
        ],
        out_shape=[
            jax.ShapeDtypeStruct((N, F), jnp.float32),
            jax.ShapeDtypeStruct((N, HD), jnp.float32),
            jax.ShapeDtypeStruct((N, HD), jnp.float32),
            jax.ShapeDtypeStruct((1, HD), jnp.float32),
        ],
        interpret=interpret,
    )(x, wall, a1t, a2t)


def _sc_scores(src_r, tgt_r, s1t, s2t, mt, z16):
    mesh = plsc.VectorSubcoreMesh(core_axis_name="c", subcore_axis_name="s")

    @functools.partial(
        pl.kernel,
        out_type=(
            jax.ShapeDtypeStruct((E, HD), jnp.float32),
            jax.ShapeDtypeStruct((N, HD), jnp.float32),
            jax.ShapeDtypeStruct((N, HD), jnp.float32),
        ),
        mesh=mesh,
        scratch_types=[
            pltpu.VMEM((NCH, CHUNK), jnp.int32),
            pltpu.VMEM((NCH, CHUNK), jnp.int32),
            pltpu.VMEM((3, CHUNK, HD), jnp.float32),
            pltpu.VMEM((3, CHUNK, HD), jnp.float32),
            pltpu.VMEM((3, CHUNK, HD), jnp.float32),
            pltpu.VMEM((HD,), jnp.float32),
            pltpu.VMEM_SHARED((N, HD), jnp.float32),
            (pltpu.SemaphoreType.DMA,) * 3,
            (pltpu.SemaphoreType.DMA,) * 3,
            (pltpu.SemaphoreType.DMA,) * 3,
        ],
        compiler_params=pltpu.CompilerParams(use_tc_tiling_on_sc=False),
    )
    def k(src_hbm, tgt_hbm, s1_hbm, s2_hbm, mt_hbm, z_hbm,
          ex_hbm, n0_hbm, n1_hbm,
          src_all, tgt_all, gs2, gt2, ex2, mt_v, norm_sh, semg, semw,
          semsc):
        cid = lax.axis_index("c")
        sid = lax.axis_index("s")
        wid = sid * NC + cid
        _node_slab_copy(sid, lambda o, n: z_hbm.at[pl.ds(0, n)],
                        lambda o, n: norm_sh.at[pl.ds(o, n)])
        pltpu.sync_copy(mt_hbm, mt_v)
        pltpu.sync_copy(src_hbm.at[wid], src_all)
        pltpu.sync_copy(tgt_hbm.at[wid], tgt_all)
        plsc.subcore_barrier()
        mt_row = mt_v[...]
        base = wid * EPT
        gs = [gs2.at[0], gs2.at[1], gs2.at[2]]
        gt = [gt2.at[0], gt2.at[1], gt2.at[2]]
        exv = [ex2.at[0], ex2.at[1], ex2.at[2]]

        def descs(slot, ci):
            return (
                pltpu.make_async_copy(s1_hbm.at[src_all.at[ci]], gs[slot],
                                      semg[slot]),
                pltpu.make_async_copy(s2_hbm.at[tgt_all.at[ci]], gt[slot],
                                      semg[slot]),
            )

        def wr_descs(slot, ci):
            off = pl.multiple_of(base + ci * CHUNK, 8)
            return (
                pltpu.make_async_copy(exv[slot], ex_hbm.at[pl.ds(off, CHUNK)],
                                      semw[slot]),
                pltpu.make_async_copy(exv[slot], norm_sh.at[tgt_all.at[ci]],
                                      semsc[slot]),
            )

        def start(slot, ci):
            d1, d2 = descs(slot, ci)
            d1.start()
            d2.start()

        def work(slot, ci):
            d1, d2 = descs(slot, ci)
            d1.wait()
            d2.wait()
            dw, dsc = wr_descs(slot, ci)

            @pl.when(jnp.asarray(ci, jnp.int32) >= 3)
            def _():
                dw.wait()
                dsc.wait()

            g_s = gs[slot]
            g_t = gt[slot]
            e_v = exv[slot]
            for i in range(CHUNK):
                s = g_s[i, :] + g_t[i, :]
                s = jnp.where(s > 0, s, ALPHA * s)
                e_v[i, :] = jnp.exp(s - mt_row)
            dw.start()
            dsc.start(add=True)

        start(0, 0)
        start(1, 1)
        ntrip = (NCH - 2) // 3

        def triple(i, carry):
            c = 3 * i
            for kk in range(3):
                work(kk, c + kk)
                start((kk + 2) % 3, c + kk + 2)
            return carry

        lax.fori_loop(0, ntrip, triple, 0)
        for c in range(3 * ntrip, NCH):
            work(c % 3, c)
            if c + 2 < NCH:
                start((c + 2) % 3, c + 2)
        for slot in range(3):
            dw, dsc = wr_descs(slot, 0)
            dw.wait()
            dsc.wait()
        plsc.subcore_barrier()

        @pl.when(cid == 0)
        def _():
            _node_slab_copy(sid, lambda o, n: norm_sh.at[pl.ds(o, n)],
                            lambda o, n: n0_hbm.at[pl.ds(o, n)])

        @pl.when(cid == 1)
        def _():
            _node_slab_copy(sid, lambda o, n: norm_sh.at[pl.ds(o, n)],
                            lambda o, n: n1_hbm.at[pl.ds(o, n)])

    return k(src_r, tgt_r, s1t, s2t, mt, z16)


def _sc_aggregate(src_r, tgt_r, ex, n0, n1, h, z128):
    mesh = plsc.VectorSubcoreMesh(core_axis_name="c", subcore_axis_name="s")

    @functools.partial(
        pl.kernel,
        out_type=(
            jax.ShapeDtypeStruct((N, F), jnp.float32),
            jax.ShapeDtypeStruct((N, F), jnp.float32),
            jax.ShapeDtypeStruct((N, HD), jnp.float32),
        ),
        mesh=mesh,
        scratch_types=[
            pltpu.VMEM((ANCH, ACHUNK), jnp.int32),
            pltpu.VMEM((ANCH, ACHUNK), jnp.int32),
            pltpu.VMEM((3, ACHUNK, F), jnp.float32),
            pltpu.VMEM((3, ACHUNK, HD), jnp.float32),
            pltpu.VMEM((3, ACHUNK, HD), jnp.float32),
            pltpu.VMEM((NB, HD), jnp.float32),
            pltpu.VMEM((NB, HD), jnp.float32),
            pltpu.VMEM_SHARED((N, F), jnp.float32),
            (pltpu.SemaphoreType.DMA,) * 3,
            (pltpu.SemaphoreType.DMA,) * 3,
            (pltpu.SemaphoreType.DMA,) * 3,
            (pltpu.SemaphoreType.DMA,) * 3,
        ],
        compiler_params=pltpu.CompilerParams(use_tc_tiling_on_sc=False),
    )
    def k(src_hbm, tgt_hbm, ex_hbm, n0_hbm, n1_hbm, h_hbm, z_hbm,
          p0_hbm, p1_hbm, rn_hbm,
          src_all, tgt_all, hb3, g03, ex3, nb0, nb1, out_sh,
          semh, semn, seme, semsc):
        cid = lax.axis_index("c")
        sid = lax.axis_index("s")
        wid = sid * NC + cid
        _node_slab_copy(sid, lambda o, n: z_hbm.at[pl.ds(0, n)],
                        lambda o, n: out_sh.at[pl.ds(o, n)])
        pltpu.sync_copy(src_hbm.at[wid], src_all)
        pltpu.sync_copy(tgt_hbm.at[wid], tgt_all)

        def rn_block(off, nrows):
            pltpu.sync_copy(n0_hbm.at[pl.ds(off, nrows)],
                            nb0.at[pl.ds(0, nrows)])
            pltpu.sync_copy(n1_hbm.at[pl.ds(off, nrows)],
                            nb1.at[pl.ds(0, nrows)])
            for r in range(nrows):
                nb0[r, :] = 1.0 / (nb0[r, :] + nb1[r, :])
            pltpu.sync_copy(nb0.at[pl.ds(0, nrows)],
                            rn_hbm.at[pl.ds(off, nrows)])

        def rn_loop(b, carry):
            rn_block(pl.multiple_of(sid * RPS + b * NB, 8), NB)
            return carry

        lax.fori_loop(0, RPS // NB, rn_loop, 0)

        @pl.when(sid == NS - 1)
        def _():
            rn_block(15 * RPS + RPS // NB * NB, RPS_LAST - RPS)

        plsc.subcore_barrier()
        base = wid * EPT
        hb = [hb3.at[0], hb3.at[1], hb3.at[2]]
        g0 = [g03.at[0], g03.at[1], g03.at[2]]
        exv = [ex3.at[0], ex3.at[1], ex3.at[2]]

        def descs(slot, ci):
            off = pl.multiple_of(base + ci * ACHUNK, 8)
            return (
                pltpu.make_async_copy(h_hbm.at[src_all.at[ci]], hb[slot],
                                      semh[slot]),
                pltpu.make_async_copy(rn_hbm.at[tgt_all.at[ci]], g0[slot],
                                      semn[slot]),
                pltpu.make_async_copy(ex_hbm.at[pl.ds(off, ACHUNK)],
                                      exv[slot], seme[slot]),
            )

        def sc_desc(slot, ci):
            return pltpu.make_async_copy(hb[slot], out_sh.at[tgt_all.at[ci]],
                                         semsc[slot])

        def start(slot, ci):
            @pl.when(jnp.asarray(ci, jnp.int32) >= 3)
            def _():
                sc_desc(slot, ci).wait()
            for d in descs(slot, ci):
                d.start()

        def work(slot, ci):
            dh, d0, de = descs(slot, ci)
            d0.wait()
            de.wait()
            dh.wait()
            e_v = exv[slot]
            g_0 = g0[slot]
            h_b = hb[slot]
            for i in range(ACHUNK):
                row = e_v[i, :] * g_0[i, :]
                for j in range(H):
                    a_b = _lane_bcast(row, j)
                    h_b[i, pl.ds(j * HD, HD)] = (
                        h_b[i, pl.ds(j * HD, HD)] * a_b)
            sc_desc(slot, ci).start(add=True)

        start(0, 0)
        start(1, 1)
        ntrip = (ANCH - 2) // 3

        def triple(i, carry):
            c = 3 * i
            for kk in range(3):
                work(kk, c + kk)
                start((kk + 2) % 3, c + kk + 2)
            return carry

        lax.fori_loop(0, ntrip, triple, 0)
        for c in range(3 * ntrip, ANCH):
            work(c % 3, c)
            if c + 2 < ANCH:
                start((c + 2) % 3, c + 2)
        for slot in range(3):
            pltpu.make_async_copy(hb[slot], out_sh.at[tgt_all.at[0]],
                                  semsc[slot]).wait()
        plsc.subcore_barrier()

        @pl.when(cid == 0)
        def _():
            _node_slab_copy(sid, lambda o, n: out_sh.at[pl.ds(o, n)],
                            lambda o, n: p0_hbm.at[pl.ds(o, n)])

        @pl.when(cid == 1)
        def _():
            _node_slab_copy(sid, lambda o, n: out_sh.at[pl.ds(o, n)],
                            lambda o, n: p1_hbm.at[pl.ds(o, n)])

    return k(src_r, tgt_r, ex, n0, n1, h, z128)


def _tc_combine_body(p0_ref, p1_ref, b_ref, o_ref):
    o_ref[...] = p0_ref[...] + p1_ref[...] + b_ref[...]


def _tc_combine(p0, p1, bias, interpret=False):
    return pl.pallas_call(
        _tc_combine_body,
        grid=(N // BN,),
        in_specs=[
            pl.BlockSpec((BN, F), lambda i: (i, 0)),
            pl.BlockSpec((BN, F), lambda i: (i, 0)),
            pl.BlockSpec((1, F), lambda i: (0, 0)),
        ],
        out_specs=pl.BlockSpec((BN, F), lambda i: (i, 0)),
        out_shape=jax.ShapeDtypeStruct((N, F), jnp.float32),
        interpret=interpret,
    )(p0, p1, bias.reshape(1, F))


def kernel(x, edge_index, W, a, bias):
    src = edge_index[0].reshape(NW, NCH, CHUNK)
    tgt = edge_index[1].reshape(NW, NCH, CHUNK)
    src_a = edge_index[0].reshape(NW, ANCH, ACHUNK)
    tgt_a = edge_index[1].reshape(NW, ANCH, ACHUNK)
    wall = jnp.transpose(W, (1, 0, 2)).reshape(F, F)
    eye8 = jnp.eye(H, dtype=jnp.float32)
    a1 = a[:, :HD, 0]
    a2 = a[:, HD:, 0]
    A1 = jnp.einsum('gk,hg->hkg', a1, eye8).reshape(F, H)
    A2 = jnp.einsum('gk,hg->hkg', a2, eye8).reshape(F, H)
    a1t = jnp.concatenate([A1, A1], axis=1)
    a2t = jnp.concatenate([A2, A2], axis=1)

    h, s1t, s2t, mt2 = _tc_head(x, wall, a1t, a2t)
    mt = mt2.reshape(HD)

    z16 = jnp.zeros((RPS_LAST, HD), jnp.float32)
    ex, n0, n1 = _sc_scores(src, tgt, s1t, s2t, mt, z16)

    z128 = jnp.zeros((RPS_LAST, F), jnp.float32)
    p0, p1, _ = _sc_aggregate(src_a, tgt_a, ex, n0, n1, h, z128)

    return _tc_combine(p0, p1, bias)
